# baseline (speedup 1.0000x reference)
import functools

import jax
import jax.numpy as jnp
from jax import lax
from jax.experimental import pallas as pl
from jax.experimental.pallas import tpu as pltpu
from jax.experimental.pallas import tpu_sc as plsc

_EMBED = 64
_NC, _NS = 2, 16
_NW = _NC * _NS
_CHUNK = 128
_SCALE = 8.0
_L = 16
_NBUF = 4

_BATCH, _SEQ = 4096, 200
_NBB = _BATCH // _CHUNK
_NCHUNK = _SEQ * _NBB
_CPW = _NCHUNK // _NW
_NGRP = _CPW // _NBUF

_mesh = plsc.VectorSubcoreMesh(
    core_axis_name="c", subcore_axis_name="s",
    num_cores=_NC, num_subcores=_NS,
)


@functools.partial(
    pl.kernel,
    out_type=jax.ShapeDtypeStruct((_SEQ, 8, _NBB, 8, _CHUNK), jnp.float32),
    mesh=_mesh,
    scratch_types=[
        pltpu.VMEM((_CPW, _CHUNK), jnp.int32),
        pltpu.VMEM((_NBUF, _CHUNK, _EMBED), jnp.float32),
        pltpu.VMEM((_NBUF, 8, 8, _CHUNK), jnp.float32),
        pltpu.SemaphoreType.DMA((_NBUF,)),
        pltpu.SemaphoreType.DMA((_NBUF,)),
    ],
    compiler_params=pltpu.CompilerParams(
        use_tc_tiling_on_sc=False, needs_layout_passes=False),
)
def _emb_lookup(x_hbm, table_hbm, out_hbm, idx_v, gbuf, wbuf, gsem, wsem):
    wid = lax.axis_index("s") * _NC + lax.axis_index("c")
    k0 = wid * _CPW
    pltpu.sync_copy(x_hbm.at[pl.ds(k0, _CPW)], idx_v)

    def fire_gather(c, b):
        pltpu.async_copy(table_hbm.at[idx_v.at[c]], gbuf.at[b], gsem.at[b])

    def wait_gather(c, b):
        pltpu.make_async_copy(table_hbm.at[idx_v.at[c]], gbuf.at[b],
                              gsem.at[b]).wait()

    def out_slice(k):
        s8, r = k // (_NBB * 8), k % (_NBB * 8)
        b_hi, s_lo = r // 8, r % 8
        return out_hbm.at[s8 * 8 + s_lo, :, b_hi]

    def fire_write(k, b):
        pltpu.async_copy(wbuf.at[b], out_slice(k), wsem.at[b])

    def wait_write(k, b):
        pltpu.make_async_copy(wbuf.at[b], out_slice(k), wsem.at[b]).wait()

    iota = lax.iota(jnp.int32, _L)

    def transpose_scale(b):
        @plsc.parallel_loop(0, _EMBED, unroll=2)
        def _col(e):
            ecol = jnp.full((_L,), 0, jnp.int32) + e
            e8, e_lo = e // 8, e % 8
            for l in range(_CHUNK // _L):
                rows = iota + (l * _L)
                v = plsc.load_gather(gbuf.at[b], [rows, ecol])
                wbuf[b, e8, e_lo, pl.ds(l * _L, _L)] = v * _SCALE

    for b in range(_NBUF):
        fire_gather(b, b)

    for b in range(_NBUF):
        wait_gather(b, b)
        transpose_scale(b)
        fire_write(k0 + b, b)
        fire_gather(_NBUF + b, b)

    def group(g, carry):
        for b in range(_NBUF):
            c = g * _NBUF + b
            wait_gather(c, b)
            wait_write(k0 + c - _NBUF, b)
            transpose_scale(b)
            fire_write(k0 + c, b)
            fire_gather(c + _NBUF, b)
        return carry

    lax.fori_loop(1, _NGRP - 1, group, 0)

    for b in range(_NBUF):
        c = (_NGRP - 1) * _NBUF + b
        wait_gather(c, b)
        wait_write(k0 + c - _NBUF, b)
        transpose_scale(b)
        fire_write(k0 + c, b)

    for b in range(_NBUF):
        wait_write(k0 + (_NGRP - 1) * _NBUF + b, b)


def kernel(x, table):
    xv = (x.astype(jnp.int32).T
          .reshape(_SEQ // 8, 8, _NBB, _CHUNK)
          .transpose(0, 2, 1, 3)
          .reshape(_NCHUNK, _CHUNK))
    o5 = _emb_lookup(xv, table)
    return (o5.transpose(2, 4, 0, 1, 3)
            .reshape(_BATCH, _SEQ, _EMBED))


# speedup vs baseline: 1.6726x; 1.6726x over previous
import functools

import jax
import jax.numpy as jnp
from jax import lax
from jax.experimental import pallas as pl
from jax.experimental.pallas import tpu as pltpu
from jax.experimental.pallas import tpu_sc as plsc

_EMBED = 64
_NC, _NS = 2, 16
_NW = _NC * _NS
_CHUNK = 128
_SCALE = 8.0
_L = 16
_NBUF = 4

_BATCH, _SEQ = 4096, 200
_NBB = _BATCH // _CHUNK
_NCHUNK = _SEQ * _NBB
_CPW = _NCHUNK // _NW
_NGRP = _CPW // _NBUF

_mesh = plsc.VectorSubcoreMesh(
    core_axis_name="c", subcore_axis_name="s",
    num_cores=_NC, num_subcores=_NS,
)


@functools.partial(
    pl.kernel,
    out_type=jax.ShapeDtypeStruct((_SEQ, 8, _NBB, 8, _CHUNK), jnp.float32),
    mesh=_mesh,
    scratch_types=[
        pltpu.VMEM((_CPW, _CHUNK), jnp.int32),
        pltpu.VMEM((_NBUF, _CHUNK, _EMBED), jnp.float32),
        pltpu.VMEM((_NBUF, 8, 8, _CHUNK + 1), jnp.float32),
        pltpu.SemaphoreType.DMA((_NBUF,)),
        pltpu.SemaphoreType.DMA((_NBUF,)),
    ],
    compiler_params=pltpu.CompilerParams(
        use_tc_tiling_on_sc=False, needs_layout_passes=False),
)
def _emb_lookup(x_hbm, table_hbm, out_hbm, idx_v, gbuf, wbuf, gsem, wsem):
    wid = lax.axis_index("s") * _NC + lax.axis_index("c")
    k0 = wid * _CPW
    pltpu.sync_copy(x_hbm.at[pl.ds(k0, _CPW)], idx_v)

    def fire_gather(c, b):
        pltpu.async_copy(table_hbm.at[idx_v.at[c]], gbuf.at[b], gsem.at[b])

    def wait_gather(c, b):
        pltpu.make_async_copy(table_hbm.at[idx_v.at[c]], gbuf.at[b],
                              gsem.at[b]).wait()

    def out_slice(k):
        s8, r = k // (_NBB * 8), k % (_NBB * 8)
        b_hi, s_lo = r // 8, r % 8
        return out_hbm.at[s8 * 8 + s_lo, :, b_hi]

    def fire_write(k, b):
        pltpu.async_copy(wbuf.at[b, :, :, pl.ds(0, _CHUNK)], out_slice(k),
                         wsem.at[b])

    def wait_write(k, b):
        pltpu.make_async_copy(wbuf.at[b, :, :, pl.ds(0, _CHUNK)],
                              out_slice(k), wsem.at[b]).wait()

    iota = lax.iota(jnp.int32, _L)
    e8v = [(iota + l * _L) // 8 for l in range(_EMBED // _L)]
    e_lov = [(iota + l * _L) % 8 for l in range(_EMBED // _L)]

    def transpose_scale(b):
        @plsc.parallel_loop(0, _CHUNK, unroll=2)
        def _row(r):
            rcol = jnp.full((_L,), 0, jnp.int32) + r
            for l in range(_EMBED // _L):
                v = gbuf[b, r, pl.ds(l * _L, _L)]
                plsc.store_scatter(wbuf.at[b], [e8v[l], e_lov[l], rcol],
                                   v * _SCALE)

    for b in range(_NBUF):
        fire_gather(b, b)

    for b in range(_NBUF):
        wait_gather(b, b)
        transpose_scale(b)
        fire_write(k0 + b, b)
        fire_gather(_NBUF + b, b)

    def group(g, carry):
        for b in range(_NBUF):
            c = g * _NBUF + b
            wait_gather(c, b)
            wait_write(k0 + c - _NBUF, b)
            transpose_scale(b)
            fire_write(k0 + c, b)
            fire_gather(c + _NBUF, b)
        return carry

    lax.fori_loop(1, _NGRP - 1, group, 0)

    for b in range(_NBUF):
        c = (_NGRP - 1) * _NBUF + b
        wait_gather(c, b)
        wait_write(k0 + c - _NBUF, b)
        transpose_scale(b)
        fire_write(k0 + c, b)

    for b in range(_NBUF):
        wait_write(k0 + (_NGRP - 1) * _NBUF + b, b)


def kernel(x, table):
    xv = (x.astype(jnp.int32).T
          .reshape(_SEQ // 8, 8, _NBB, _CHUNK)
          .transpose(0, 2, 1, 3)
          .reshape(_NCHUNK, _CHUNK))
    o5 = _emb_lookup(xv, table)
    return (o5.transpose(2, 4, 0, 1, 3)
            .reshape(_BATCH, _SEQ, _EMBED))
